# Initial kernel scaffold; baseline (speedup 1.0000x reference)
import jax
import jax.numpy as jnp
from jax.experimental import pallas as pl


def kernel(feats, inner_w, inner_b, cf_w1, cf_b1, cf_g1, cf_be1, cf_w2, cf_b2, mlp_w1, mlp_b1, mlp_g, mlp_be, mlp_w2, mlp_b2):
    raise NotImplementedError("write your pallas kernel here")


# speedup vs baseline: 5.5039x; 5.5039x over previous
import functools

import jax
import jax.numpy as jnp
from jax import lax
from jax.experimental import pallas as pl
from jax.experimental.pallas import tpu as pltpu
from jax.experimental.pallas import tpu_sc as plsc

K = 8
B = 4
C = 128
N = 2048
TN = 256
TP = 128
M = B * N * K
EPS = 1e-5
_PREC = lax.Precision.HIGHEST
_DIST_PREC = lax.Precision.DEFAULT


def _knn_body(f_ref, xt_ref, iw_ref, idx_ref, w_ref):
    b = pl.program_id(0)
    x = f_ref[0]
    xt = xt_ref[0]
    iw = iw_ref[...]

    s_full = jnp.sum(x * iw, axis=0, keepdims=True)
    xx_full = jnp.sum(x * x, axis=0, keepdims=True)
    xx_row = jnp.sum(xt * xt, axis=1, keepdims=True)
    inner = jnp.dot(xt, x, preferred_element_type=jnp.float32,
                    precision=_DIST_PREC)
    dist = 2.0 * inner - xx_row - xx_full

    col = lax.broadcasted_iota(jnp.int32, (TN, N), 1)
    idx_cols = []
    sg_cols = []
    for _ in range(K):
        m = jnp.max(dist, axis=1, keepdims=True)
        is_max = dist == m
        idx_j = jnp.min(jnp.where(is_max, col, N), axis=1, keepdims=True)
        hit = col == idx_j
        sg_j = jnp.sum(jnp.where(hit, s_full, 0.0), axis=1, keepdims=True)
        dist = jnp.where(hit, -jnp.inf, dist)
        idx_cols.append(idx_j)
        sg_cols.append(sg_j)
    idx = jnp.concatenate(idx_cols, axis=1)
    sg = jnp.concatenate(sg_cols, axis=1)

    logits = -sg
    lmax = jnp.max(logits, axis=1, keepdims=True)
    e = jnp.exp(logits - lmax)
    w = e / jnp.sum(e, axis=1, keepdims=True)

    idx_ref[0] = idx + b * N
    w_ref[0] = w


def _knn(feats, xt3, iw2):
    grid = (B, N // TN)
    return pl.pallas_call(
        _knn_body,
        grid=grid,
        in_specs=[
            pl.BlockSpec((1, C, N), lambda b, t: (b, 0, 0)),
            pl.BlockSpec((1, TN, C), lambda b, t: (b, t, 0)),
            pl.BlockSpec((C, 1), lambda b, t: (0, 0)),
        ],
        out_specs=[
            pl.BlockSpec((1, TN, K), lambda b, t: (b, t, 0)),
            pl.BlockSpec((1, TN, K), lambda b, t: (b, t, 0)),
        ],
        out_shape=[
            jax.ShapeDtypeStruct((B, N, K), jnp.int32),
            jax.ShapeDtypeStruct((B, N, K), jnp.float32),
        ],
    )(feats, xt3, iw2)


_NUM_SC = 2
_NUM_SUBCORES = 16
_NW = _NUM_SC * _NUM_SUBCORES
_ROWS_PER_W = M // _NW
_CHUNK = 128
_NCHUNK = _ROWS_PER_W // _CHUNK


def _gather_body(table_hbm, idx_hbm, out_hbm, idx_v, rows_a, rows_b, sem_a,
                 sem_b):
    wid = lax.axis_index("c") * _NUM_SUBCORES + lax.axis_index("s")
    base = wid * _ROWS_PER_W
    pltpu.sync_copy(idx_hbm.at[pl.ds(base, _ROWS_PER_W)], idx_v)

    bufs = (rows_a, rows_b)
    sems = (sem_a, sem_b)

    def start(c):
        pltpu.async_copy(
            table_hbm.at[idx_v.at[pl.ds(c * _CHUNK, _CHUNK)]],
            bufs[c % 2], sems[c % 2])

    start(0)
    for c in range(_NCHUNK):
        if c + 1 < _NCHUNK:
            start(c + 1)
        pltpu.make_async_copy(
            table_hbm.at[idx_v.at[pl.ds(c * _CHUNK, _CHUNK)]],
            bufs[c % 2], sems[c % 2]).wait()
        pltpu.sync_copy(bufs[c % 2],
                        out_hbm.at[pl.ds(base + c * _CHUNK, _CHUNK)])


def _sc_gather(table, fidx):
    mesh = plsc.VectorSubcoreMesh(core_axis_name="c", subcore_axis_name="s")
    k = pl.kernel(
        _gather_body,
        out_type=jax.ShapeDtypeStruct((M, C), jnp.float32),
        mesh=mesh,
        scratch_types=[
            pltpu.VMEM((_ROWS_PER_W,), jnp.int32),
            pltpu.VMEM((_CHUNK, C), jnp.float32),
            pltpu.VMEM((_CHUNK, C), jnp.float32),
            pltpu.SemaphoreType.DMA,
            pltpu.SemaphoreType.DMA,
        ],
    )
    return k(table, fidx)


def _conv1_body(f_ref, xt_ref, w_ref, w1a_ref, w1b_ref, b1_ref, h1_ref,
                st_ref):
    i = pl.program_id(0)
    xt = xt_ref[...]
    a = jnp.dot(xt, w1a_ref[...], preferred_element_type=jnp.float32,
                precision=_PREC) + b1_ref[...]

    @pl.when(i == 0)
    def _():
        st_ref[...] = jnp.zeros_like(st_ref)

    s1 = jnp.zeros((1, 2 * C), jnp.float32)
    s2 = jnp.zeros((1, 2 * C), jnp.float32)
    for j in range(K):
        fj = f_ref[:, j, :]
        wj = w_ref[:, j, :]
        hj = a + jnp.dot(fj * wj, w1b_ref[...],
                         preferred_element_type=jnp.float32,
                         precision=_PREC)
        h1_ref[:, j, :] = hj
        s1 = s1 + jnp.sum(hj, axis=0, keepdims=True)
        s2 = s2 + jnp.sum(hj * hj, axis=0, keepdims=True)
    st_ref[...] += jnp.concatenate([s1, s2], axis=0)


def _conv1(F3, xt_rows, w3, w1aT, w1bT, b1):
    grid = (B * N // TP,)
    return pl.pallas_call(
        _conv1_body,
        grid=grid,
        in_specs=[
            pl.BlockSpec((TP, K, C), lambda i: (i, 0, 0)),
            pl.BlockSpec((TP, C), lambda i: (i, 0)),
            pl.BlockSpec((TP, K, 1), lambda i: (i, 0, 0)),
            pl.BlockSpec((C, 2 * C), lambda i: (0, 0)),
            pl.BlockSpec((C, 2 * C), lambda i: (0, 0)),
            pl.BlockSpec((1, 2 * C), lambda i: (0, 0)),
        ],
        out_specs=[
            pl.BlockSpec((TP, K, 2 * C), lambda i: (i, 0, 0)),
            pl.BlockSpec((2, 2 * C), lambda i: (0, 0)),
        ],
        out_shape=[
            jax.ShapeDtypeStruct((B * N, K, 2 * C), jnp.float32),
            jax.ShapeDtypeStruct((2, 2 * C), jnp.float32),
        ],
    )(F3, xt_rows, w3, w1aT, w1bT, b1)


def _mid_body(h1_ref, st_ref, g_ref, be_ref, wc_ref, bc_ref, q_ref, st2_ref):
    i = pl.program_id(0)
    st = st_ref[...]
    mean = st[0:1, :] * (1.0 / M)
    var = st[1:2, :] * (1.0 / M) - mean * mean
    inv = lax.rsqrt(var + EPS)
    scale = g_ref[...] * inv
    shift = be_ref[...] - mean * scale

    h = h1_ref[...].reshape(TP * K, 2 * C)
    h = jnp.maximum(h * scale + shift, 0.0)
    q = jnp.dot(h, wc_ref[...], preferred_element_type=jnp.float32,
                precision=_PREC) + bc_ref[...]

    @pl.when(i == 0)
    def _():
        st2_ref[...] = jnp.zeros_like(st2_ref)

    s1 = jnp.sum(q, axis=0, keepdims=True)
    s2 = jnp.sum(q * q, axis=0, keepdims=True)
    st2_ref[...] += jnp.concatenate([s1, s2], axis=0)
    q_ref[...] = q


def _mid(h1, st1, g1, be1, wcT, bc):
    grid = (M // (TP * K),)
    return pl.pallas_call(
        _mid_body,
        grid=grid,
        in_specs=[
            pl.BlockSpec((TP, K, 2 * C), lambda i: (i, 0, 0)),
            pl.BlockSpec((2, 2 * C), lambda i: (0, 0)),
            pl.BlockSpec((1, 2 * C), lambda i: (0, 0)),
            pl.BlockSpec((1, 2 * C), lambda i: (0, 0)),
            pl.BlockSpec((2 * C, C), lambda i: (0, 0)),
            pl.BlockSpec((1, C), lambda i: (0, 0)),
        ],
        out_specs=[
            pl.BlockSpec((TP * K, C), lambda i: (i, 0)),
            pl.BlockSpec((2, C), lambda i: (0, 0)),
        ],
        out_shape=[
            jax.ShapeDtypeStruct((M, C), jnp.float32),
            jax.ShapeDtypeStruct((2, C), jnp.float32),
        ],
    )(h1, st1, g1, be1, wcT, bc)


def _final_body(q_ref, st_ref, g_ref, be_ref, w2_ref, b2_ref, o_ref):
    st = st_ref[...]
    mean = st[0:1, :] * (1.0 / M)
    var = st[1:2, :] * (1.0 / M) - mean * mean
    inv = lax.rsqrt(var + EPS)
    scale = g_ref[...] * inv
    shift = be_ref[...] - mean * scale

    q = jnp.maximum(q_ref[...] * scale + shift, 0.0)
    z = jnp.dot(q, w2_ref[...], preferred_element_type=jnp.float32,
                precision=_PREC) + b2_ref[...]
    z3 = z.reshape(TP, K, C)
    m = z3[:, 0, :]
    for j in range(1, K):
        m = jnp.maximum(m, z3[:, j, :])
    o_ref[...] = m


def _final(q, st2, g2, be2, w2T, b2):
    grid = (M // (TP * K),)
    return pl.pallas_call(
        _final_body,
        grid=grid,
        in_specs=[
            pl.BlockSpec((TP * K, C), lambda i: (i, 0)),
            pl.BlockSpec((2, C), lambda i: (0, 0)),
            pl.BlockSpec((1, C), lambda i: (0, 0)),
            pl.BlockSpec((1, C), lambda i: (0, 0)),
            pl.BlockSpec((C, C), lambda i: (0, 0)),
            pl.BlockSpec((1, C), lambda i: (0, 0)),
        ],
        out_specs=pl.BlockSpec((TP, C), lambda i: (i, 0)),
        out_shape=jax.ShapeDtypeStruct((B * N, C), jnp.float32),
    )(q, st2, g2, be2, w2T, b2)


def kernel(feats, inner_w, inner_b, cf_w1, cf_b1, cf_g1, cf_be1, cf_w2,
           cf_b2, mlp_w1, mlp_b1, mlp_g, mlp_be, mlp_w2, mlp_b2):
    del inner_b
    xt3 = jnp.transpose(feats, (0, 2, 1))
    xt_rows = xt3.reshape(B * N, C)
    iw2 = inner_w.reshape(C, 1)

    idx, w = _knn(feats, xt3, iw2)
    fidx = idx.reshape(M)
    F = _sc_gather(xt_rows, fidx)
    F3 = F.reshape(B * N, K, C)
    w3 = w.reshape(B * N, K, 1)

    w1aT = jnp.transpose(cf_w1[:, :C])
    w1bT = jnp.transpose(cf_w1[:, C:])
    b1 = cf_b1.reshape(1, 2 * C)
    h1, st1 = _conv1(F3, xt_rows, w3, w1aT, w1bT, b1)

    wc = jnp.dot(mlp_w1, cf_w2, precision=_PREC)
    bc = (jnp.dot(mlp_w1, cf_b2, precision=_PREC) + mlp_b1).reshape(1, C)
    q, st2 = _mid(h1, st1, cf_g1.reshape(1, 2 * C), cf_be1.reshape(1, 2 * C),
                  jnp.transpose(wc), bc)

    rows = _final(q, st2, mlp_g.reshape(1, C), mlp_be.reshape(1, C),
                  jnp.transpose(mlp_w2), mlp_b2.reshape(1, C))
    out = jnp.transpose(rows.reshape(B, N, C), (0, 2, 1))[:, :, :, None]
    return out


# speedup vs baseline: 7.1280x; 1.2951x over previous
import functools

import jax
import jax.numpy as jnp
from jax import lax
from jax.experimental import pallas as pl
from jax.experimental.pallas import tpu as pltpu
from jax.experimental.pallas import tpu_sc as plsc

K = 8
B = 4
C = 128
N = 2048
TN = 256
TP = 128
M = B * N * K
EPS = 1e-5
_PREC = lax.Precision.HIGHEST
_DIST_PREC = lax.Precision.DEFAULT
_CPREC = lax.Precision.DEFAULT


def _knn_body(f_ref, xt_ref, idx_ref):
    b = pl.program_id(0)
    x = f_ref[0]
    xt = xt_ref[0]

    xx_full = jnp.sum(x * x, axis=0, keepdims=True)
    xx_row = jnp.sum(xt * xt, axis=1, keepdims=True)
    inner = jnp.dot(xt, x, preferred_element_type=jnp.float32,
                    precision=_DIST_PREC)
    dist = 2.0 * inner - xx_row - xx_full

    col = lax.broadcasted_iota(jnp.int32, (TN, N), 1)
    idx_cols = []
    for _ in range(K):
        m = jnp.max(dist, axis=1, keepdims=True)
        is_max = dist == m
        idx_j = jnp.min(jnp.where(is_max, col, N), axis=1, keepdims=True)
        dist = jnp.where(col == idx_j, -jnp.inf, dist)
        idx_cols.append(idx_j)
    idx = jnp.concatenate(idx_cols, axis=1)
    idx_ref[0] = idx + b * N


def _knn(feats, xt3):
    grid = (B, N // TN)
    return pl.pallas_call(
        _knn_body,
        grid=grid,
        in_specs=[
            pl.BlockSpec((1, C, N), lambda b, t: (b, 0, 0)),
            pl.BlockSpec((1, TN, C), lambda b, t: (b, t, 0)),
        ],
        out_specs=pl.BlockSpec((1, TN, K), lambda b, t: (b, t, 0)),
        out_shape=jax.ShapeDtypeStruct((B, N, K), jnp.int32),
    )(feats, xt3)


_NUM_SC = 2
_NUM_SUBCORES = 16
_NW = _NUM_SC * _NUM_SUBCORES
_ROWS_PER_W = M // _NW
_CHUNK = 128
_NCHUNK = _ROWS_PER_W // _CHUNK


def _gather_body(table_hbm, idx_hbm, out_hbm, idx_v, rows_a, rows_b, sem_a,
                 sem_b):
    wid = lax.axis_index("c") * _NUM_SUBCORES + lax.axis_index("s")
    base = wid * _ROWS_PER_W
    pltpu.sync_copy(idx_hbm.at[pl.ds(base, _ROWS_PER_W)], idx_v)

    bufs = (rows_a, rows_b)
    sems = (sem_a, sem_b)

    def start(c):
        pltpu.async_copy(
            table_hbm.at[idx_v.at[pl.ds(c * _CHUNK, _CHUNK)]],
            bufs[c % 2], sems[c % 2])

    start(0)
    for c in range(_NCHUNK):
        if c + 1 < _NCHUNK:
            start(c + 1)
        pltpu.make_async_copy(
            table_hbm.at[idx_v.at[pl.ds(c * _CHUNK, _CHUNK)]],
            bufs[c % 2], sems[c % 2]).wait()
        pltpu.sync_copy(bufs[c % 2],
                        out_hbm.at[pl.ds(base + c * _CHUNK, _CHUNK)])


def _sc_gather(table, fidx):
    mesh = plsc.VectorSubcoreMesh(core_axis_name="c", subcore_axis_name="s")
    k = pl.kernel(
        _gather_body,
        out_type=jax.ShapeDtypeStruct((M, C), jnp.float32),
        mesh=mesh,
        scratch_types=[
            pltpu.VMEM((_ROWS_PER_W,), jnp.int32),
            pltpu.VMEM((_CHUNK, C), jnp.float32),
            pltpu.VMEM((_CHUNK, C), jnp.float32),
            pltpu.SemaphoreType.DMA,
            pltpu.SemaphoreType.DMA,
        ],
    )
    return k(table, fidx)


def _conv1_body(f_ref, xt_ref, iw_ref, w1a_ref, w1b_ref, b1_ref, h1_ref,
                st_ref):
    i = pl.program_id(0)
    xt = xt_ref[...]
    a = jnp.dot(xt, w1a_ref[...], preferred_element_type=jnp.float32,
                precision=_CPREC) + b1_ref[...]

    ff = f_ref[...].reshape(TP * K, C)
    sg = jnp.dot(ff, iw_ref[...], preferred_element_type=jnp.float32,
                 precision=_PREC)
    sg3 = sg.reshape(TP, K, 1)
    mn = sg3[:, 0, :]
    for j in range(1, K):
        mn = jnp.minimum(mn, sg3[:, j, :])
    e3 = jnp.exp(mn[:, None, :] - sg3)
    den = e3[:, 0, :]
    for j in range(1, K):
        den = den + e3[:, j, :]
    w3 = e3 * (1.0 / den)[:, None, :]

    wf = w3.reshape(TP * K, 1)
    hb = jnp.dot(ff * wf, w1b_ref[...],
                 preferred_element_type=jnp.float32,
                 precision=_CPREC)
    h = hb.reshape(TP, K, 2 * C) + a[:, None, :]
    h1_ref[...] = h

    hf = h.reshape(TP * K, 2 * C)
    s1 = jnp.sum(hf, axis=0, keepdims=True)
    s2 = jnp.sum(hf * hf, axis=0, keepdims=True)

    @pl.when(i == 0)
    def _():
        st_ref[...] = jnp.zeros_like(st_ref)

    st_ref[...] += jnp.concatenate([s1, s2], axis=0)


def _conv1(F3, xt_rows, iw2, w1aT, w1bT, b1):
    grid = (B * N // TP,)
    return pl.pallas_call(
        _conv1_body,
        grid=grid,
        in_specs=[
            pl.BlockSpec((TP, K, C), lambda i: (i, 0, 0)),
            pl.BlockSpec((TP, C), lambda i: (i, 0)),
            pl.BlockSpec((C, 1), lambda i: (0, 0)),
            pl.BlockSpec((C, 2 * C), lambda i: (0, 0)),
            pl.BlockSpec((C, 2 * C), lambda i: (0, 0)),
            pl.BlockSpec((1, 2 * C), lambda i: (0, 0)),
        ],
        out_specs=[
            pl.BlockSpec((TP, K, 2 * C), lambda i: (i, 0, 0)),
            pl.BlockSpec((2, 2 * C), lambda i: (0, 0)),
        ],
        out_shape=[
            jax.ShapeDtypeStruct((B * N, K, 2 * C), jnp.float32),
            jax.ShapeDtypeStruct((2, 2 * C), jnp.float32),
        ],
    )(F3, xt_rows, iw2, w1aT, w1bT, b1)


def _mid_body(h1_ref, st_ref, g_ref, be_ref, wc_ref, bc_ref, q_ref, st2_ref):
    i = pl.program_id(0)
    st = st_ref[...]
    mean = st[0:1, :] * (1.0 / M)
    var = st[1:2, :] * (1.0 / M) - mean * mean
    inv = lax.rsqrt(var + EPS)
    scale = g_ref[...] * inv
    shift = be_ref[...] - mean * scale

    h = h1_ref[...].reshape(TP * K, 2 * C)
    h = jnp.maximum(h * scale + shift, 0.0)
    q = jnp.dot(h, wc_ref[...], preferred_element_type=jnp.float32,
                precision=_CPREC) + bc_ref[...]

    @pl.when(i == 0)
    def _():
        st2_ref[...] = jnp.zeros_like(st2_ref)

    s1 = jnp.sum(q, axis=0, keepdims=True)
    s2 = jnp.sum(q * q, axis=0, keepdims=True)
    st2_ref[...] += jnp.concatenate([s1, s2], axis=0)
    q_ref[...] = q


def _mid(h1, st1, g1, be1, wcT, bc):
    grid = (M // (TP * K),)
    return pl.pallas_call(
        _mid_body,
        grid=grid,
        in_specs=[
            pl.BlockSpec((TP, K, 2 * C), lambda i: (i, 0, 0)),
            pl.BlockSpec((2, 2 * C), lambda i: (0, 0)),
            pl.BlockSpec((1, 2 * C), lambda i: (0, 0)),
            pl.BlockSpec((1, 2 * C), lambda i: (0, 0)),
            pl.BlockSpec((2 * C, C), lambda i: (0, 0)),
            pl.BlockSpec((1, C), lambda i: (0, 0)),
        ],
        out_specs=[
            pl.BlockSpec((TP * K, C), lambda i: (i, 0)),
            pl.BlockSpec((2, C), lambda i: (0, 0)),
        ],
        out_shape=[
            jax.ShapeDtypeStruct((M, C), jnp.float32),
            jax.ShapeDtypeStruct((2, C), jnp.float32),
        ],
    )(h1, st1, g1, be1, wcT, bc)


def _final_body(q_ref, st_ref, g_ref, be_ref, w2_ref, b2_ref, o_ref):
    st = st_ref[...]
    mean = st[0:1, :] * (1.0 / M)
    var = st[1:2, :] * (1.0 / M) - mean * mean
    inv = lax.rsqrt(var + EPS)
    scale = g_ref[...] * inv
    shift = be_ref[...] - mean * scale

    q = jnp.maximum(q_ref[...] * scale + shift, 0.0)
    z = jnp.dot(q, w2_ref[...], preferred_element_type=jnp.float32,
                precision=_CPREC) + b2_ref[...]
    z3 = z.reshape(TP, K, C)
    m = z3[:, 0, :]
    for j in range(1, K):
        m = jnp.maximum(m, z3[:, j, :])
    o_ref[...] = m


def _final(q, st2, g2, be2, w2T, b2):
    grid = (M // (TP * K),)
    return pl.pallas_call(
        _final_body,
        grid=grid,
        in_specs=[
            pl.BlockSpec((TP * K, C), lambda i: (i, 0)),
            pl.BlockSpec((2, C), lambda i: (0, 0)),
            pl.BlockSpec((1, C), lambda i: (0, 0)),
            pl.BlockSpec((1, C), lambda i: (0, 0)),
            pl.BlockSpec((C, C), lambda i: (0, 0)),
            pl.BlockSpec((1, C), lambda i: (0, 0)),
        ],
        out_specs=pl.BlockSpec((TP, C), lambda i: (i, 0)),
        out_shape=jax.ShapeDtypeStruct((B * N, C), jnp.float32),
    )(q, st2, g2, be2, w2T, b2)


def kernel(feats, inner_w, inner_b, cf_w1, cf_b1, cf_g1, cf_be1, cf_w2,
           cf_b2, mlp_w1, mlp_b1, mlp_g, mlp_be, mlp_w2, mlp_b2):
    del inner_b
    xt3 = jnp.transpose(feats, (0, 2, 1))
    xt_rows = xt3.reshape(B * N, C)
    iw2 = inner_w.reshape(C, 1)

    idx = _knn(feats, xt3)
    fidx = idx.reshape(M)
    F = _sc_gather(xt_rows, fidx)
    F3 = F.reshape(B * N, K, C)

    w1aT = jnp.transpose(cf_w1[:, :C])
    w1bT = jnp.transpose(cf_w1[:, C:])
    b1 = cf_b1.reshape(1, 2 * C)
    h1, st1 = _conv1(F3, xt_rows, iw2, w1aT, w1bT, b1)

    wc = jnp.dot(mlp_w1, cf_w2, precision=_PREC)
    bc = (jnp.dot(mlp_w1, cf_b2, precision=_PREC) + mlp_b1).reshape(1, C)
    q, st2 = _mid(h1, st1, cf_g1.reshape(1, 2 * C), cf_be1.reshape(1, 2 * C),
                  jnp.transpose(wc), bc)

    rows = _final(q, st2, mlp_g.reshape(1, C), mlp_be.reshape(1, C),
                  jnp.transpose(mlp_w2), mlp_b2.reshape(1, C))
    out = jnp.transpose(rows.reshape(B, N, C), (0, 2, 1))[:, :, :, None]
    return out


# speedup vs baseline: 7.6514x; 1.0734x over previous
import functools

import jax
import jax.numpy as jnp
from jax import lax
from jax.experimental import pallas as pl
from jax.experimental.pallas import tpu as pltpu
from jax.experimental.pallas import tpu_sc as plsc

K = 8
B = 4
C = 128
N = 2048
TN = 256
TP = 128
M = B * N * K
EPS = 1e-5
_PREC = lax.Precision.HIGHEST
_DIST_PREC = lax.Precision.DEFAULT
_CPREC = lax.Precision.DEFAULT


def _knn_body(f_ref, xt_ref, idx_ref):
    b = pl.program_id(0)
    x = f_ref[0]
    xt = xt_ref[0]

    xx_full = jnp.sum(x * x, axis=0, keepdims=True)
    xx_row = jnp.sum(xt * xt, axis=1, keepdims=True)
    inner = jnp.dot(xt, x, preferred_element_type=jnp.float32,
                    precision=_DIST_PREC)
    dist = 2.0 * inner - xx_row - xx_full

    col = lax.broadcasted_iota(jnp.int32, (TN, N), 1)
    idx_cols = []
    for _ in range(K):
        idx_j = jnp.argmax(dist, axis=1, keepdims=True)
        dist = jnp.where(col == idx_j, -jnp.inf, dist)
        idx_cols.append(idx_j)
    idx = jnp.concatenate(idx_cols, axis=1)
    idx_ref[0] = idx + b * N


def _knn(feats, xt3):
    grid = (B, N // TN)
    return pl.pallas_call(
        _knn_body,
        grid=grid,
        in_specs=[
            pl.BlockSpec((1, C, N), lambda b, t: (b, 0, 0)),
            pl.BlockSpec((1, TN, C), lambda b, t: (b, t, 0)),
        ],
        out_specs=pl.BlockSpec((1, TN, K), lambda b, t: (b, t, 0)),
        out_shape=jax.ShapeDtypeStruct((B, N, K), jnp.int32),
    )(feats, xt3)


_NUM_SC = 2
_NUM_SUBCORES = 16
_NW = _NUM_SC * _NUM_SUBCORES
_ROWS_PER_W = M // _NW
_CHUNK = 128
_NCHUNK = _ROWS_PER_W // _CHUNK


def _gather_body(table_hbm, idx_hbm, out_hbm, idx_v, rows_a, rows_b, sem_a,
                 sem_b):
    wid = lax.axis_index("c") * _NUM_SUBCORES + lax.axis_index("s")
    base = wid * _ROWS_PER_W
    pltpu.sync_copy(idx_hbm.at[pl.ds(base, _ROWS_PER_W)], idx_v)

    bufs = (rows_a, rows_b)
    sems = (sem_a, sem_b)

    def start(c):
        pltpu.async_copy(
            table_hbm.at[idx_v.at[pl.ds(c * _CHUNK, _CHUNK)]],
            bufs[c % 2], sems[c % 2])

    start(0)
    for c in range(_NCHUNK):
        if c + 1 < _NCHUNK:
            start(c + 1)
        pltpu.make_async_copy(
            table_hbm.at[idx_v.at[pl.ds(c * _CHUNK, _CHUNK)]],
            bufs[c % 2], sems[c % 2]).wait()
        pltpu.sync_copy(bufs[c % 2],
                        out_hbm.at[pl.ds(base + c * _CHUNK, _CHUNK)])


def _sc_gather(table, fidx):
    mesh = plsc.VectorSubcoreMesh(core_axis_name="c", subcore_axis_name="s")
    k = pl.kernel(
        _gather_body,
        out_type=jax.ShapeDtypeStruct((M, C), jnp.float32),
        mesh=mesh,
        scratch_types=[
            pltpu.VMEM((_ROWS_PER_W,), jnp.int32),
            pltpu.VMEM((_CHUNK, C), jnp.float32),
            pltpu.VMEM((_CHUNK, C), jnp.float32),
            pltpu.SemaphoreType.DMA,
            pltpu.SemaphoreType.DMA,
        ],
    )
    return k(table, fidx)


def _conv1_body(f_ref, xt_ref, iw_ref, w1a_ref, w1b_ref, b1_ref, h1_ref,
                st_ref):
    i = pl.program_id(0)
    xt = xt_ref[...]
    a = jnp.dot(xt, w1a_ref[...], preferred_element_type=jnp.float32,
                precision=_CPREC) + b1_ref[...]

    ff = f_ref[...].reshape(TP * K, C)
    sg = jnp.dot(ff, iw_ref[...], preferred_element_type=jnp.float32,
                 precision=_PREC)
    sg3 = sg.reshape(TP, K, 1)
    mn = sg3[:, 0, :]
    for j in range(1, K):
        mn = jnp.minimum(mn, sg3[:, j, :])
    e3 = jnp.exp(mn[:, None, :] - sg3)
    den = e3[:, 0, :]
    for j in range(1, K):
        den = den + e3[:, j, :]
    w3 = e3 * (1.0 / den)[:, None, :]

    wf = w3.reshape(TP * K, 1)
    hb = jnp.dot(ff * wf, w1b_ref[...],
                 preferred_element_type=jnp.float32,
                 precision=_CPREC)
    h = hb.reshape(TP, K, 2 * C) + a[:, None, :]
    h1_ref[...] = h

    hf = h.reshape(TP * K, 2 * C)
    s1 = jnp.sum(hf, axis=0, keepdims=True)
    s2 = jnp.sum(hf * hf, axis=0, keepdims=True)

    @pl.when(i == 0)
    def _():
        st_ref[...] = jnp.zeros_like(st_ref)

    st_ref[...] += jnp.concatenate([s1, s2], axis=0)


def _conv1(F3, xt_rows, iw2, w1aT, w1bT, b1):
    grid = (B * N // TP,)
    return pl.pallas_call(
        _conv1_body,
        grid=grid,
        in_specs=[
            pl.BlockSpec((TP, K, C), lambda i: (i, 0, 0)),
            pl.BlockSpec((TP, C), lambda i: (i, 0)),
            pl.BlockSpec((C, 1), lambda i: (0, 0)),
            pl.BlockSpec((C, 2 * C), lambda i: (0, 0)),
            pl.BlockSpec((C, 2 * C), lambda i: (0, 0)),
            pl.BlockSpec((1, 2 * C), lambda i: (0, 0)),
        ],
        out_specs=[
            pl.BlockSpec((TP, K, 2 * C), lambda i: (i, 0, 0)),
            pl.BlockSpec((2, 2 * C), lambda i: (0, 0)),
        ],
        out_shape=[
            jax.ShapeDtypeStruct((B * N, K, 2 * C), jnp.float32),
            jax.ShapeDtypeStruct((2, 2 * C), jnp.float32),
        ],
    )(F3, xt_rows, iw2, w1aT, w1bT, b1)


def _mid_body(h1_ref, st_ref, g_ref, be_ref, wc_ref, bc_ref, q_ref, st2_ref):
    i = pl.program_id(0)
    st = st_ref[...]
    mean = st[0:1, :] * (1.0 / M)
    var = st[1:2, :] * (1.0 / M) - mean * mean
    inv = lax.rsqrt(var + EPS)
    scale = g_ref[...] * inv
    shift = be_ref[...] - mean * scale

    h = h1_ref[...].reshape(TP * K, 2 * C)
    h = jnp.maximum(h * scale + shift, 0.0)
    q = jnp.dot(h, wc_ref[...], preferred_element_type=jnp.float32,
                precision=_CPREC) + bc_ref[...]

    @pl.when(i == 0)
    def _():
        st2_ref[...] = jnp.zeros_like(st2_ref)

    s1 = jnp.sum(q, axis=0, keepdims=True)
    s2 = jnp.sum(q * q, axis=0, keepdims=True)
    st2_ref[...] += jnp.concatenate([s1, s2], axis=0)
    q_ref[...] = q


def _mid(h1, st1, g1, be1, wcT, bc):
    grid = (M // (TP * K),)
    return pl.pallas_call(
        _mid_body,
        grid=grid,
        in_specs=[
            pl.BlockSpec((TP, K, 2 * C), lambda i: (i, 0, 0)),
            pl.BlockSpec((2, 2 * C), lambda i: (0, 0)),
            pl.BlockSpec((1, 2 * C), lambda i: (0, 0)),
            pl.BlockSpec((1, 2 * C), lambda i: (0, 0)),
            pl.BlockSpec((2 * C, C), lambda i: (0, 0)),
            pl.BlockSpec((1, C), lambda i: (0, 0)),
        ],
        out_specs=[
            pl.BlockSpec((TP * K, C), lambda i: (i, 0)),
            pl.BlockSpec((2, C), lambda i: (0, 0)),
        ],
        out_shape=[
            jax.ShapeDtypeStruct((M, C), jnp.float32),
            jax.ShapeDtypeStruct((2, C), jnp.float32),
        ],
    )(h1, st1, g1, be1, wcT, bc)


def _final_body(q_ref, st_ref, g_ref, be_ref, w2_ref, b2_ref, o_ref):
    st = st_ref[...]
    mean = st[0:1, :] * (1.0 / M)
    var = st[1:2, :] * (1.0 / M) - mean * mean
    inv = lax.rsqrt(var + EPS)
    scale = g_ref[...] * inv
    shift = be_ref[...] - mean * scale

    q = jnp.maximum(q_ref[...] * scale + shift, 0.0)
    z = jnp.dot(q, w2_ref[...], preferred_element_type=jnp.float32,
                precision=_CPREC) + b2_ref[...]
    z3 = z.reshape(TP, K, C)
    m = z3[:, 0, :]
    for j in range(1, K):
        m = jnp.maximum(m, z3[:, j, :])
    o_ref[...] = m


def _final(q, st2, g2, be2, w2T, b2):
    grid = (M // (TP * K),)
    return pl.pallas_call(
        _final_body,
        grid=grid,
        in_specs=[
            pl.BlockSpec((TP * K, C), lambda i: (i, 0)),
            pl.BlockSpec((2, C), lambda i: (0, 0)),
            pl.BlockSpec((1, C), lambda i: (0, 0)),
            pl.BlockSpec((1, C), lambda i: (0, 0)),
            pl.BlockSpec((C, C), lambda i: (0, 0)),
            pl.BlockSpec((1, C), lambda i: (0, 0)),
        ],
        out_specs=pl.BlockSpec((TP, C), lambda i: (i, 0)),
        out_shape=jax.ShapeDtypeStruct((B * N, C), jnp.float32),
    )(q, st2, g2, be2, w2T, b2)


def kernel(feats, inner_w, inner_b, cf_w1, cf_b1, cf_g1, cf_be1, cf_w2,
           cf_b2, mlp_w1, mlp_b1, mlp_g, mlp_be, mlp_w2, mlp_b2):
    del inner_b
    xt3 = jnp.transpose(feats, (0, 2, 1))
    xt_rows = xt3.reshape(B * N, C)
    iw2 = inner_w.reshape(C, 1)

    idx = _knn(feats, xt3)
    fidx = idx.reshape(M)
    F = _sc_gather(xt_rows, fidx)
    F3 = F.reshape(B * N, K, C)

    w1aT = jnp.transpose(cf_w1[:, :C])
    w1bT = jnp.transpose(cf_w1[:, C:])
    b1 = cf_b1.reshape(1, 2 * C)
    h1, st1 = _conv1(F3, xt_rows, iw2, w1aT, w1bT, b1)

    wc = jnp.dot(mlp_w1, cf_w2, precision=_PREC)
    bc = (jnp.dot(mlp_w1, cf_b2, precision=_PREC) + mlp_b1).reshape(1, C)
    q, st2 = _mid(h1, st1, cf_g1.reshape(1, 2 * C), cf_be1.reshape(1, 2 * C),
                  jnp.transpose(wc), bc)

    rows = _final(q, st2, mlp_g.reshape(1, C), mlp_be.reshape(1, C),
                  jnp.transpose(mlp_w2), mlp_b2.reshape(1, C))
    out = jnp.transpose(rows.reshape(B, N, C), (0, 2, 1))[:, :, :, None]
    return out


# speedup vs baseline: 7.9717x; 1.0419x over previous
import functools

import jax
import jax.numpy as jnp
from jax import lax
from jax.experimental import pallas as pl
from jax.experimental.pallas import tpu as pltpu
from jax.experimental.pallas import tpu_sc as plsc

K = 8
B = 4
C = 128
N = 2048
TN = 256
TP = 128
M = B * N * K
EPS = 1e-5
_PREC = lax.Precision.HIGHEST
_DIST_PREC = lax.Precision.DEFAULT
_CPREC = lax.Precision.DEFAULT


def _knn_body(f_ref, xt_ref, idx_ref, *, boff):
    b = pl.program_id(0)
    x = f_ref[0]
    xt = xt_ref[0]

    xx_full = jnp.sum(x * x, axis=0, keepdims=True)
    xx_row = jnp.sum(xt * xt, axis=1, keepdims=True)
    inner = jnp.dot(xt, x, preferred_element_type=jnp.float32,
                    precision=_DIST_PREC)
    dist = 2.0 * inner - xx_row - xx_full

    col = lax.broadcasted_iota(jnp.int32, (TN, N), 1)
    idx_cols = []
    for _ in range(K):
        idx_j = jnp.argmax(dist, axis=1, keepdims=True)
        dist = jnp.where(col == idx_j, -jnp.inf, dist)
        idx_cols.append(idx_j)
    idx = jnp.concatenate(idx_cols, axis=1)
    idx_ref[0] = idx + (b + boff) * N


def _knn(feats, xt3, boff):
    nb = feats.shape[0]
    grid = (nb, N // TN)
    return pl.pallas_call(
        functools.partial(_knn_body, boff=boff),
        grid=grid,
        in_specs=[
            pl.BlockSpec((1, C, N), lambda b, t: (b, 0, 0)),
            pl.BlockSpec((1, TN, C), lambda b, t: (b, t, 0)),
        ],
        out_specs=pl.BlockSpec((1, TN, K), lambda b, t: (b, t, 0)),
        out_shape=jax.ShapeDtypeStruct((nb, N, K), jnp.int32),
    )(feats, xt3)


_NUM_SC = 2
_NUM_SUBCORES = 16
_NW = _NUM_SC * _NUM_SUBCORES
_CHUNK = 128


def _gather_body(table_hbm, idx_hbm, out_hbm, idx_v, rows_a, rows_b, sem_a,
                 sem_b, *, rows_per_w):
    wid = lax.axis_index("c") * _NUM_SUBCORES + lax.axis_index("s")
    base = wid * rows_per_w
    pltpu.sync_copy(idx_hbm.at[pl.ds(base, rows_per_w)], idx_v)

    bufs = (rows_a, rows_b)
    sems = (sem_a, sem_b)
    nchunk = rows_per_w // _CHUNK

    def start(c):
        pltpu.async_copy(
            table_hbm.at[idx_v.at[pl.ds(c * _CHUNK, _CHUNK)]],
            bufs[c % 2], sems[c % 2])

    start(0)
    for c in range(nchunk):
        if c + 1 < nchunk:
            start(c + 1)
        pltpu.make_async_copy(
            table_hbm.at[idx_v.at[pl.ds(c * _CHUNK, _CHUNK)]],
            bufs[c % 2], sems[c % 2]).wait()
        pltpu.sync_copy(bufs[c % 2],
                        out_hbm.at[pl.ds(base + c * _CHUNK, _CHUNK)])


def _sc_gather(table, fidx):
    m = fidx.shape[0]
    rows_per_w = m // _NW
    mesh = plsc.VectorSubcoreMesh(core_axis_name="c", subcore_axis_name="s")
    k = pl.kernel(
        functools.partial(_gather_body, rows_per_w=rows_per_w),
        out_type=jax.ShapeDtypeStruct((m, C), jnp.float32),
        mesh=mesh,
        scratch_types=[
            pltpu.VMEM((rows_per_w,), jnp.int32),
            pltpu.VMEM((_CHUNK, C), jnp.float32),
            pltpu.VMEM((_CHUNK, C), jnp.float32),
            pltpu.SemaphoreType.DMA,
            pltpu.SemaphoreType.DMA,
        ],
    )
    return k(table, fidx)


def _conv1_body(f_ref, xt_ref, iw_ref, w1a_ref, w1b_ref, b1_ref, h1_ref,
                st_ref):
    i = pl.program_id(0)
    xt = xt_ref[...]
    a = jnp.dot(xt, w1a_ref[...], preferred_element_type=jnp.float32,
                precision=_CPREC) + b1_ref[...]

    ff = f_ref[...].reshape(TP * K, C)
    sg = jnp.dot(ff, iw_ref[...], preferred_element_type=jnp.float32,
                 precision=_PREC)
    sg3 = sg.reshape(TP, K, 1)
    mn = sg3[:, 0, :]
    for j in range(1, K):
        mn = jnp.minimum(mn, sg3[:, j, :])
    e3 = jnp.exp(mn[:, None, :] - sg3)
    den = e3[:, 0, :]
    for j in range(1, K):
        den = den + e3[:, j, :]
    w3 = e3 * (1.0 / den)[:, None, :]

    wf = w3.reshape(TP * K, 1)
    hb = jnp.dot(ff * wf, w1b_ref[...],
                 preferred_element_type=jnp.float32,
                 precision=_CPREC)
    h = hb.reshape(TP, K, 2 * C) + a[:, None, :]
    h1_ref[...] = h

    hf = h.reshape(TP * K, 2 * C)
    s1 = jnp.sum(hf, axis=0, keepdims=True)
    s2 = jnp.sum(hf * hf, axis=0, keepdims=True)

    @pl.when(i == 0)
    def _():
        st_ref[...] = jnp.zeros_like(st_ref)

    st_ref[...] += jnp.concatenate([s1, s2], axis=0)


def _conv1(F3, xt_rows, iw2, w1aT, w1bT, b1):
    npts = F3.shape[0]
    grid = (npts // TP,)
    return pl.pallas_call(
        _conv1_body,
        grid=grid,
        in_specs=[
            pl.BlockSpec((TP, K, C), lambda i: (i, 0, 0)),
            pl.BlockSpec((TP, C), lambda i: (i, 0)),
            pl.BlockSpec((C, 1), lambda i: (0, 0)),
            pl.BlockSpec((C, 2 * C), lambda i: (0, 0)),
            pl.BlockSpec((C, 2 * C), lambda i: (0, 0)),
            pl.BlockSpec((1, 2 * C), lambda i: (0, 0)),
        ],
        out_specs=[
            pl.BlockSpec((TP, K, 2 * C), lambda i: (i, 0, 0)),
            pl.BlockSpec((2, 2 * C), lambda i: (0, 0)),
        ],
        out_shape=[
            jax.ShapeDtypeStruct((npts, K, 2 * C), jnp.float32),
            jax.ShapeDtypeStruct((2, 2 * C), jnp.float32),
        ],
    )(F3, xt_rows, iw2, w1aT, w1bT, b1)


def _mid_body(h1_ref, st_ref, g_ref, be_ref, wc_ref, bc_ref, q_ref, st2_ref):
    i = pl.program_id(0)
    st = st_ref[...]
    mean = st[0:1, :] * (1.0 / M)
    var = st[1:2, :] * (1.0 / M) - mean * mean
    inv = lax.rsqrt(var + EPS)
    scale = g_ref[...] * inv
    shift = be_ref[...] - mean * scale

    h = h1_ref[...].reshape(TP * K, 2 * C)
    h = jnp.maximum(h * scale + shift, 0.0)
    q = jnp.dot(h, wc_ref[...], preferred_element_type=jnp.float32,
                precision=_CPREC) + bc_ref[...]

    @pl.when(i == 0)
    def _():
        st2_ref[...] = jnp.zeros_like(st2_ref)

    s1 = jnp.sum(q, axis=0, keepdims=True)
    s2 = jnp.sum(q * q, axis=0, keepdims=True)
    st2_ref[...] += jnp.concatenate([s1, s2], axis=0)
    q_ref[...] = q


def _mid(h1, st1, g1, be1, wcT, bc):
    npts = h1.shape[0]
    grid = (npts // TP,)
    return pl.pallas_call(
        _mid_body,
        grid=grid,
        in_specs=[
            pl.BlockSpec((TP, K, 2 * C), lambda i: (i, 0, 0)),
            pl.BlockSpec((2, 2 * C), lambda i: (0, 0)),
            pl.BlockSpec((1, 2 * C), lambda i: (0, 0)),
            pl.BlockSpec((1, 2 * C), lambda i: (0, 0)),
            pl.BlockSpec((2 * C, C), lambda i: (0, 0)),
            pl.BlockSpec((1, C), lambda i: (0, 0)),
        ],
        out_specs=[
            pl.BlockSpec((TP * K, C), lambda i: (i, 0)),
            pl.BlockSpec((2, C), lambda i: (0, 0)),
        ],
        out_shape=[
            jax.ShapeDtypeStruct((npts * K, C), jnp.float32),
            jax.ShapeDtypeStruct((2, C), jnp.float32),
        ],
    )(h1, st1, g1, be1, wcT, bc)


def _final_body(q_ref, st_ref, g_ref, be_ref, w2_ref, b2_ref, o_ref):
    st = st_ref[...]
    mean = st[0:1, :] * (1.0 / M)
    var = st[1:2, :] * (1.0 / M) - mean * mean
    inv = lax.rsqrt(var + EPS)
    scale = g_ref[...] * inv
    shift = be_ref[...] - mean * scale

    q = jnp.maximum(q_ref[...] * scale + shift, 0.0)
    z = jnp.dot(q, w2_ref[...], preferred_element_type=jnp.float32,
                precision=_CPREC) + b2_ref[...]
    z3 = z.reshape(TP, K, C)
    m = z3[:, 0, :]
    for j in range(1, K):
        m = jnp.maximum(m, z3[:, j, :])
    o_ref[...] = m


def _final(q, st2, g2, be2, w2T, b2):
    nrows = q.shape[0]
    grid = (nrows // (TP * K),)
    return pl.pallas_call(
        _final_body,
        grid=grid,
        in_specs=[
            pl.BlockSpec((TP * K, C), lambda i: (i, 0)),
            pl.BlockSpec((2, C), lambda i: (0, 0)),
            pl.BlockSpec((1, C), lambda i: (0, 0)),
            pl.BlockSpec((1, C), lambda i: (0, 0)),
            pl.BlockSpec((C, C), lambda i: (0, 0)),
            pl.BlockSpec((1, C), lambda i: (0, 0)),
        ],
        out_specs=pl.BlockSpec((TP, C), lambda i: (i, 0)),
        out_shape=jax.ShapeDtypeStruct((nrows // K, C), jnp.float32),
    )(q, st2, g2, be2, w2T, b2)


def kernel(feats, inner_w, inner_b, cf_w1, cf_b1, cf_g1, cf_be1, cf_w2,
           cf_b2, mlp_w1, mlp_b1, mlp_g, mlp_be, mlp_w2, mlp_b2):
    del inner_b
    xt3 = jnp.transpose(feats, (0, 2, 1))
    xt_rows = xt3.reshape(B * N, C)
    iw2 = inner_w.reshape(C, 1)

    w1aT = jnp.transpose(cf_w1[:, :C])
    w1bT = jnp.transpose(cf_w1[:, C:])
    b1 = cf_b1.reshape(1, 2 * C)
    wc = jnp.dot(mlp_w1, cf_w2, precision=_PREC)
    bc = (jnp.dot(mlp_w1, cf_b2, precision=_PREC) + mlp_b1).reshape(1, C)
    wcT = jnp.transpose(wc)

    hb = B // 2
    hpts = hb * N
    idxs = [_knn(feats[i * hb:(i + 1) * hb], xt3[i * hb:(i + 1) * hb],
                 i * hb) for i in range(2)]
    Fs = [_sc_gather(xt_rows, idx.reshape(hpts * K)) for idx in idxs]

    cres = [_conv1(F.reshape(hpts, K, C),
                   xt_rows[i * hpts:(i + 1) * hpts], iw2, w1aT, w1bT, b1)
            for i, F in enumerate(Fs)]
    st1 = cres[0][1] + cres[1][1]

    g1r, be1r = cf_g1.reshape(1, 2 * C), cf_be1.reshape(1, 2 * C)
    mres = [_mid(h1, st1, g1r, be1r, wcT, bc) for h1, _ in cres]
    st2 = mres[0][1] + mres[1][1]

    g2r, be2r = mlp_g.reshape(1, C), mlp_be.reshape(1, C)
    w2T, b2r = jnp.transpose(mlp_w2), mlp_b2.reshape(1, C)
    rows = jnp.concatenate(
        [_final(q, st2, g2r, be2r, w2T, b2r) for q, _ in mres], axis=0)
    out = jnp.transpose(rows.reshape(B, N, C), (0, 2, 1))[:, :, :, None]
    return out


# speedup vs baseline: 7.9816x; 1.0012x over previous
import functools

import jax
import jax.numpy as jnp
from jax import lax
from jax.experimental import pallas as pl
from jax.experimental.pallas import tpu as pltpu
from jax.experimental.pallas import tpu_sc as plsc

K = 8
B = 4
C = 128
N = 2048
TN = 256
TP = 128
M = B * N * K
EPS = 1e-5
_PREC = lax.Precision.HIGHEST
_DIST_PREC = lax.Precision.DEFAULT
_CPREC = lax.Precision.DEFAULT


def _knn_body(f_ref, xt_ref, idx_ref, *, boff):
    b = pl.program_id(0)
    x = f_ref[0]
    xt = xt_ref[0]

    xx_full = jnp.sum(x * x, axis=0, keepdims=True)
    xx_row = jnp.sum(xt * xt, axis=1, keepdims=True)
    inner = jnp.dot(xt, x, preferred_element_type=jnp.float32,
                    precision=_DIST_PREC)
    dist = 2.0 * inner - xx_row - xx_full

    col = lax.broadcasted_iota(jnp.int32, (TN, N), 1)
    idx_cols = []
    for _ in range(K):
        idx_j = jnp.argmax(dist, axis=1, keepdims=True)
        dist = jnp.where(col == idx_j, -jnp.inf, dist)
        idx_cols.append(idx_j)
    idx = jnp.concatenate(idx_cols, axis=1)
    idx_ref[0] = idx + (b + boff) * N


def _knn(feats, xt3, boff):
    nb = feats.shape[0]
    grid = (nb, N // TN)
    return pl.pallas_call(
        functools.partial(_knn_body, boff=boff),
        grid=grid,
        in_specs=[
            pl.BlockSpec((1, C, N), lambda b, t: (b, 0, 0)),
            pl.BlockSpec((1, TN, C), lambda b, t: (b, t, 0)),
        ],
        out_specs=pl.BlockSpec((1, TN, K), lambda b, t: (b, t, 0)),
        out_shape=jax.ShapeDtypeStruct((nb, N, K), jnp.int32),
    )(feats, xt3)


_NUM_SC = 2
_NUM_SUBCORES = 16
_NW = _NUM_SC * _NUM_SUBCORES
_CHUNK = 128


def _gather_body(table_hbm, idx_hbm, out_hbm, idx_v, rows_a, rows_b, gsem_a,
                 gsem_b, ssem_a, ssem_b, *, rows_per_w):
    wid = lax.axis_index("c") * _NUM_SUBCORES + lax.axis_index("s")
    base = wid * rows_per_w
    pltpu.sync_copy(idx_hbm.at[pl.ds(base, rows_per_w)], idx_v)

    bufs = (rows_a, rows_b)
    gsems = (gsem_a, gsem_b)
    ssems = (ssem_a, ssem_b)
    nchunk = rows_per_w // _CHUNK

    def gather(c):
        return (table_hbm.at[idx_v.at[pl.ds(c * _CHUNK, _CHUNK)]],
                bufs[c % 2], gsems[c % 2])

    def scatter(c):
        return (bufs[c % 2], out_hbm.at[pl.ds(base + c * _CHUNK, _CHUNK)],
                ssems[c % 2])

    pltpu.async_copy(*gather(0))
    for c in range(nchunk):
        if c + 1 < nchunk:
            if c >= 1:
                pltpu.make_async_copy(*scatter(c - 1)).wait()
            pltpu.async_copy(*gather(c + 1))
        pltpu.make_async_copy(*gather(c)).wait()
        pltpu.async_copy(*scatter(c))
    pltpu.make_async_copy(*scatter(nchunk - 2)).wait()
    pltpu.make_async_copy(*scatter(nchunk - 1)).wait()


def _sc_gather(table, fidx):
    m = fidx.shape[0]
    rows_per_w = m // _NW
    mesh = plsc.VectorSubcoreMesh(core_axis_name="c", subcore_axis_name="s")
    k = pl.kernel(
        functools.partial(_gather_body, rows_per_w=rows_per_w),
        out_type=jax.ShapeDtypeStruct((m, C), jnp.float32),
        mesh=mesh,
        scratch_types=[
            pltpu.VMEM((rows_per_w,), jnp.int32),
            pltpu.VMEM((_CHUNK, C), jnp.float32),
            pltpu.VMEM((_CHUNK, C), jnp.float32),
            pltpu.SemaphoreType.DMA,
            pltpu.SemaphoreType.DMA,
            pltpu.SemaphoreType.DMA,
            pltpu.SemaphoreType.DMA,
        ],
    )
    return k(table, fidx)


def _conv1_body(f_ref, xt_ref, iw_ref, w1a_ref, w1b_ref, b1_ref, h1_ref,
                st_ref):
    i = pl.program_id(0)
    xt = xt_ref[...]
    a = jnp.dot(xt, w1a_ref[...], preferred_element_type=jnp.float32,
                precision=_CPREC) + b1_ref[...]

    ff = f_ref[...].reshape(TP * K, C)
    sg = jnp.dot(ff, iw_ref[...], preferred_element_type=jnp.float32,
                 precision=_PREC)
    sg3 = sg.reshape(TP, K, 1)
    mn = sg3[:, 0, :]
    for j in range(1, K):
        mn = jnp.minimum(mn, sg3[:, j, :])
    e3 = jnp.exp(mn[:, None, :] - sg3)
    den = e3[:, 0, :]
    for j in range(1, K):
        den = den + e3[:, j, :]
    w3 = e3 * (1.0 / den)[:, None, :]

    wf = w3.reshape(TP * K, 1)
    hb = jnp.dot(ff * wf, w1b_ref[...],
                 preferred_element_type=jnp.float32,
                 precision=_CPREC)
    h = hb.reshape(TP, K, 2 * C) + a[:, None, :]
    h1_ref[...] = h

    hf = h.reshape(TP * K, 2 * C)
    s1 = jnp.sum(hf, axis=0, keepdims=True)
    s2 = jnp.sum(hf * hf, axis=0, keepdims=True)

    @pl.when(i == 0)
    def _():
        st_ref[...] = jnp.zeros_like(st_ref)

    st_ref[...] += jnp.concatenate([s1, s2], axis=0)


def _conv1(F3, xt_rows, iw2, w1aT, w1bT, b1):
    npts = F3.shape[0]
    grid = (npts // TP,)
    return pl.pallas_call(
        _conv1_body,
        grid=grid,
        in_specs=[
            pl.BlockSpec((TP, K, C), lambda i: (i, 0, 0)),
            pl.BlockSpec((TP, C), lambda i: (i, 0)),
            pl.BlockSpec((C, 1), lambda i: (0, 0)),
            pl.BlockSpec((C, 2 * C), lambda i: (0, 0)),
            pl.BlockSpec((C, 2 * C), lambda i: (0, 0)),
            pl.BlockSpec((1, 2 * C), lambda i: (0, 0)),
        ],
        out_specs=[
            pl.BlockSpec((TP, K, 2 * C), lambda i: (i, 0, 0)),
            pl.BlockSpec((2, 2 * C), lambda i: (0, 0)),
        ],
        out_shape=[
            jax.ShapeDtypeStruct((npts, K, 2 * C), jnp.float32),
            jax.ShapeDtypeStruct((2, 2 * C), jnp.float32),
        ],
    )(F3, xt_rows, iw2, w1aT, w1bT, b1)


def _mid_body(h1_ref, st_ref, g_ref, be_ref, wc_ref, bc_ref, q_ref, st2_ref):
    i = pl.program_id(0)
    st = st_ref[...]
    mean = st[0:1, :] * (1.0 / M)
    var = st[1:2, :] * (1.0 / M) - mean * mean
    inv = lax.rsqrt(var + EPS)
    scale = g_ref[...] * inv
    shift = be_ref[...] - mean * scale

    h = h1_ref[...].reshape(TP * K, 2 * C)
    h = jnp.maximum(h * scale + shift, 0.0)
    q = jnp.dot(h, wc_ref[...], preferred_element_type=jnp.float32,
                precision=_CPREC) + bc_ref[...]

    @pl.when(i == 0)
    def _():
        st2_ref[...] = jnp.zeros_like(st2_ref)

    s1 = jnp.sum(q, axis=0, keepdims=True)
    s2 = jnp.sum(q * q, axis=0, keepdims=True)
    st2_ref[...] += jnp.concatenate([s1, s2], axis=0)
    q_ref[...] = q


def _mid(h1, st1, g1, be1, wcT, bc):
    npts = h1.shape[0]
    grid = (npts // TP,)
    return pl.pallas_call(
        _mid_body,
        grid=grid,
        in_specs=[
            pl.BlockSpec((TP, K, 2 * C), lambda i: (i, 0, 0)),
            pl.BlockSpec((2, 2 * C), lambda i: (0, 0)),
            pl.BlockSpec((1, 2 * C), lambda i: (0, 0)),
            pl.BlockSpec((1, 2 * C), lambda i: (0, 0)),
            pl.BlockSpec((2 * C, C), lambda i: (0, 0)),
            pl.BlockSpec((1, C), lambda i: (0, 0)),
        ],
        out_specs=[
            pl.BlockSpec((TP * K, C), lambda i: (i, 0)),
            pl.BlockSpec((2, C), lambda i: (0, 0)),
        ],
        out_shape=[
            jax.ShapeDtypeStruct((npts * K, C), jnp.float32),
            jax.ShapeDtypeStruct((2, C), jnp.float32),
        ],
    )(h1, st1, g1, be1, wcT, bc)


def _final_body(q_ref, st_ref, g_ref, be_ref, w2_ref, b2_ref, o_ref):
    st = st_ref[...]
    mean = st[0:1, :] * (1.0 / M)
    var = st[1:2, :] * (1.0 / M) - mean * mean
    inv = lax.rsqrt(var + EPS)
    scale = g_ref[...] * inv
    shift = be_ref[...] - mean * scale

    q = jnp.maximum(q_ref[...] * scale + shift, 0.0)
    z = jnp.dot(q, w2_ref[...], preferred_element_type=jnp.float32,
                precision=_CPREC) + b2_ref[...]
    z3 = z.reshape(TP, K, C)
    m = z3[:, 0, :]
    for j in range(1, K):
        m = jnp.maximum(m, z3[:, j, :])
    o_ref[...] = m


def _final(q, st2, g2, be2, w2T, b2):
    nrows = q.shape[0]
    grid = (nrows // (TP * K),)
    return pl.pallas_call(
        _final_body,
        grid=grid,
        in_specs=[
            pl.BlockSpec((TP * K, C), lambda i: (i, 0)),
            pl.BlockSpec((2, C), lambda i: (0, 0)),
            pl.BlockSpec((1, C), lambda i: (0, 0)),
            pl.BlockSpec((1, C), lambda i: (0, 0)),
            pl.BlockSpec((C, C), lambda i: (0, 0)),
            pl.BlockSpec((1, C), lambda i: (0, 0)),
        ],
        out_specs=pl.BlockSpec((TP, C), lambda i: (i, 0)),
        out_shape=jax.ShapeDtypeStruct((nrows // K, C), jnp.float32),
    )(q, st2, g2, be2, w2T, b2)


def kernel(feats, inner_w, inner_b, cf_w1, cf_b1, cf_g1, cf_be1, cf_w2,
           cf_b2, mlp_w1, mlp_b1, mlp_g, mlp_be, mlp_w2, mlp_b2):
    del inner_b
    xt3 = jnp.transpose(feats, (0, 2, 1))
    xt_rows = xt3.reshape(B * N, C)
    iw2 = inner_w.reshape(C, 1)

    w1aT = jnp.transpose(cf_w1[:, :C])
    w1bT = jnp.transpose(cf_w1[:, C:])
    b1 = cf_b1.reshape(1, 2 * C)
    wc = jnp.dot(mlp_w1, cf_w2, precision=_PREC)
    bc = (jnp.dot(mlp_w1, cf_b2, precision=_PREC) + mlp_b1).reshape(1, C)
    wcT = jnp.transpose(wc)

    hb = B // 2
    hpts = hb * N
    idxs = [_knn(feats[i * hb:(i + 1) * hb], xt3[i * hb:(i + 1) * hb],
                 i * hb) for i in range(2)]
    Fs = [_sc_gather(xt_rows, idx.reshape(hpts * K)) for idx in idxs]

    cres = [_conv1(F.reshape(hpts, K, C),
                   xt_rows[i * hpts:(i + 1) * hpts], iw2, w1aT, w1bT, b1)
            for i, F in enumerate(Fs)]
    st1 = cres[0][1] + cres[1][1]

    g1r, be1r = cf_g1.reshape(1, 2 * C), cf_be1.reshape(1, 2 * C)
    mres = [_mid(h1, st1, g1r, be1r, wcT, bc) for h1, _ in cres]
    st2 = mres[0][1] + mres[1][1]

    g2r, be2r = mlp_g.reshape(1, C), mlp_be.reshape(1, C)
    w2T, b2r = jnp.transpose(mlp_w2), mlp_b2.reshape(1, C)
    rows = jnp.concatenate(
        [_final(q, st2, g2r, be2r, w2T, b2r) for q, _ in mres], axis=0)
    out = jnp.transpose(rows.reshape(B, N, C), (0, 2, 1))[:, :, :, None]
    return out


# speedup vs baseline: 8.1623x; 1.0226x over previous
import functools

import jax
import jax.numpy as jnp
from jax import lax
from jax.experimental import pallas as pl
from jax.experimental.pallas import tpu as pltpu
from jax.experimental.pallas import tpu_sc as plsc

K = 8
B = 4
C = 128
N = 2048
TN = 256
TP = 128
M = B * N * K
EPS = 1e-5
_PREC = lax.Precision.HIGHEST
_DIST_PREC = lax.Precision.DEFAULT
_CPREC = lax.Precision.DEFAULT


def _knn_body(f_ref, xt_ref, idx_ref, *, boff):
    b = pl.program_id(0)
    x = f_ref[0]
    xt = xt_ref[0]

    xx_full = jnp.sum(x * x, axis=0, keepdims=True)
    xx_row = jnp.sum(xt * xt, axis=1, keepdims=True)
    inner = jnp.dot(xt, x, preferred_element_type=jnp.float32,
                    precision=_DIST_PREC)
    dist = 2.0 * inner - xx_row - xx_full

    col = lax.broadcasted_iota(jnp.int32, (TN, N), 1)
    idx_cols = []
    for _ in range(K):
        idx_j = jnp.argmax(dist, axis=1, keepdims=True)
        dist = jnp.where(col == idx_j, -jnp.inf, dist)
        idx_cols.append(idx_j)
    idx = jnp.concatenate(idx_cols, axis=1)
    idx_ref[0] = idx + (b + boff) * N


def _knn(feats, xt3, boff):
    nb = feats.shape[0]
    grid = (nb, N // TN)
    return pl.pallas_call(
        functools.partial(_knn_body, boff=boff),
        grid=grid,
        in_specs=[
            pl.BlockSpec((1, C, N), lambda b, t: (b, 0, 0)),
            pl.BlockSpec((1, TN, C), lambda b, t: (b, t, 0)),
        ],
        out_specs=pl.BlockSpec((1, TN, K), lambda b, t: (b, t, 0)),
        out_shape=jax.ShapeDtypeStruct((nb, N, K), jnp.int32),
    )(feats, xt3)


_NUM_SC = 2
_NUM_SUBCORES = 16
_NW = _NUM_SC * _NUM_SUBCORES
_CHUNK = 128


def _gather_body(table_hbm, idx_hbm, out_hbm, idx_v, rows_a, rows_b, gsem_a,
                 gsem_b, ssem_a, ssem_b, *, rows_per_w):
    wid = lax.axis_index("c") * _NUM_SUBCORES + lax.axis_index("s")
    base = wid * rows_per_w
    pltpu.sync_copy(idx_hbm.at[pl.ds(base, rows_per_w)], idx_v)

    bufs = (rows_a, rows_b)
    gsems = (gsem_a, gsem_b)
    ssems = (ssem_a, ssem_b)
    nchunk = rows_per_w // _CHUNK

    def gather(c):
        return (table_hbm.at[idx_v.at[pl.ds(c * _CHUNK, _CHUNK)]],
                bufs[c % 2], gsems[c % 2])

    def scatter(c):
        return (bufs[c % 2], out_hbm.at[pl.ds(base + c * _CHUNK, _CHUNK)],
                ssems[c % 2])

    pltpu.async_copy(*gather(0))
    for c in range(nchunk):
        if c + 1 < nchunk:
            if c >= 1:
                pltpu.make_async_copy(*scatter(c - 1)).wait()
            pltpu.async_copy(*gather(c + 1))
        pltpu.make_async_copy(*gather(c)).wait()
        pltpu.async_copy(*scatter(c))
    pltpu.make_async_copy(*scatter(nchunk - 2)).wait()
    pltpu.make_async_copy(*scatter(nchunk - 1)).wait()


def _sc_gather(table, fidx):
    m = fidx.shape[0]
    rows_per_w = m // _NW
    mesh = plsc.VectorSubcoreMesh(core_axis_name="c", subcore_axis_name="s")
    k = pl.kernel(
        functools.partial(_gather_body, rows_per_w=rows_per_w),
        out_type=jax.ShapeDtypeStruct((m, C), jnp.float32),
        mesh=mesh,
        scratch_types=[
            pltpu.VMEM((rows_per_w,), jnp.int32),
            pltpu.VMEM((_CHUNK, C), jnp.float32),
            pltpu.VMEM((_CHUNK, C), jnp.float32),
            pltpu.SemaphoreType.DMA,
            pltpu.SemaphoreType.DMA,
            pltpu.SemaphoreType.DMA,
            pltpu.SemaphoreType.DMA,
        ],
    )
    return k(table, fidx)


def _conv1_body(f_ref, xt_ref, iw_ref, w1a_ref, w1b_ref, b1_ref, h1_ref,
                st_ref):
    i = pl.program_id(0)
    xt = xt_ref[...]
    a = jnp.dot(xt, w1a_ref[...], preferred_element_type=jnp.float32,
                precision=_CPREC) + b1_ref[...]

    ff = f_ref[...].reshape(TP * K, C)
    sg = jnp.dot(ff, iw_ref[...], preferred_element_type=jnp.float32,
                 precision=_PREC)
    sg3 = sg.reshape(TP, K, 1)
    mn = sg3[:, 0, :]
    for j in range(1, K):
        mn = jnp.minimum(mn, sg3[:, j, :])
    e3 = jnp.exp(mn[:, None, :] - sg3)
    den = e3[:, 0, :]
    for j in range(1, K):
        den = den + e3[:, j, :]
    w3 = e3 * (1.0 / den)[:, None, :]

    wf = w3.reshape(TP * K, 1)
    hb = jnp.dot(ff * wf, w1b_ref[...],
                 preferred_element_type=jnp.float32,
                 precision=_CPREC)
    h = hb.reshape(TP, K, 2 * C) + a[:, None, :]
    h1_ref[...] = h.astype(jnp.bfloat16)

    hf = h.reshape(TP * K, 2 * C)
    s1 = jnp.sum(hf, axis=0, keepdims=True)
    s2 = jnp.sum(hf * hf, axis=0, keepdims=True)

    @pl.when(i == 0)
    def _():
        st_ref[...] = jnp.zeros_like(st_ref)

    st_ref[...] += jnp.concatenate([s1, s2], axis=0)


def _conv1(F3, xt_rows, iw2, w1aT, w1bT, b1):
    npts = F3.shape[0]
    grid = (npts // TP,)
    return pl.pallas_call(
        _conv1_body,
        grid=grid,
        in_specs=[
            pl.BlockSpec((TP, K, C), lambda i: (i, 0, 0)),
            pl.BlockSpec((TP, C), lambda i: (i, 0)),
            pl.BlockSpec((C, 1), lambda i: (0, 0)),
            pl.BlockSpec((C, 2 * C), lambda i: (0, 0)),
            pl.BlockSpec((C, 2 * C), lambda i: (0, 0)),
            pl.BlockSpec((1, 2 * C), lambda i: (0, 0)),
        ],
        out_specs=[
            pl.BlockSpec((TP, K, 2 * C), lambda i: (i, 0, 0)),
            pl.BlockSpec((2, 2 * C), lambda i: (0, 0)),
        ],
        out_shape=[
            jax.ShapeDtypeStruct((npts, K, 2 * C), jnp.bfloat16),
            jax.ShapeDtypeStruct((2, 2 * C), jnp.float32),
        ],
    )(F3, xt_rows, iw2, w1aT, w1bT, b1)


def _mid_body(h1_ref, st_ref, g_ref, be_ref, wc_ref, bc_ref, q_ref, st2_ref):
    i = pl.program_id(0)
    st = st_ref[...]
    mean = st[0:1, :] * (1.0 / M)
    var = st[1:2, :] * (1.0 / M) - mean * mean
    inv = lax.rsqrt(var + EPS)
    scale = g_ref[...] * inv
    shift = be_ref[...] - mean * scale

    h = h1_ref[...].astype(jnp.float32).reshape(TP * K, 2 * C)
    h = jnp.maximum(h * scale + shift, 0.0)
    q = jnp.dot(h, wc_ref[...], preferred_element_type=jnp.float32,
                precision=_CPREC) + bc_ref[...]

    @pl.when(i == 0)
    def _():
        st2_ref[...] = jnp.zeros_like(st2_ref)

    s1 = jnp.sum(q, axis=0, keepdims=True)
    s2 = jnp.sum(q * q, axis=0, keepdims=True)
    st2_ref[...] += jnp.concatenate([s1, s2], axis=0)
    q_ref[...] = q


def _mid(h1, st1, g1, be1, wcT, bc):
    npts = h1.shape[0]
    grid = (npts // TP,)
    return pl.pallas_call(
        _mid_body,
        grid=grid,
        in_specs=[
            pl.BlockSpec((TP, K, 2 * C), lambda i: (i, 0, 0)),
            pl.BlockSpec((2, 2 * C), lambda i: (0, 0)),
            pl.BlockSpec((1, 2 * C), lambda i: (0, 0)),
            pl.BlockSpec((1, 2 * C), lambda i: (0, 0)),
            pl.BlockSpec((2 * C, C), lambda i: (0, 0)),
            pl.BlockSpec((1, C), lambda i: (0, 0)),
        ],
        out_specs=[
            pl.BlockSpec((TP * K, C), lambda i: (i, 0)),
            pl.BlockSpec((2, C), lambda i: (0, 0)),
        ],
        out_shape=[
            jax.ShapeDtypeStruct((npts * K, C), jnp.float32),
            jax.ShapeDtypeStruct((2, C), jnp.float32),
        ],
    )(h1, st1, g1, be1, wcT, bc)


def _final_body(q_ref, st_ref, g_ref, be_ref, w2_ref, b2_ref, o_ref):
    st = st_ref[...]
    mean = st[0:1, :] * (1.0 / M)
    var = st[1:2, :] * (1.0 / M) - mean * mean
    inv = lax.rsqrt(var + EPS)
    scale = g_ref[...] * inv
    shift = be_ref[...] - mean * scale

    q = jnp.maximum(q_ref[...] * scale + shift, 0.0)
    z = jnp.dot(q, w2_ref[...], preferred_element_type=jnp.float32,
                precision=_CPREC) + b2_ref[...]
    z3 = z.reshape(TP, K, C)
    m = z3[:, 0, :]
    for j in range(1, K):
        m = jnp.maximum(m, z3[:, j, :])
    o_ref[...] = m


def _final(q, st2, g2, be2, w2T, b2):
    nrows = q.shape[0]
    grid = (nrows // (TP * K),)
    return pl.pallas_call(
        _final_body,
        grid=grid,
        in_specs=[
            pl.BlockSpec((TP * K, C), lambda i: (i, 0)),
            pl.BlockSpec((2, C), lambda i: (0, 0)),
            pl.BlockSpec((1, C), lambda i: (0, 0)),
            pl.BlockSpec((1, C), lambda i: (0, 0)),
            pl.BlockSpec((C, C), lambda i: (0, 0)),
            pl.BlockSpec((1, C), lambda i: (0, 0)),
        ],
        out_specs=pl.BlockSpec((TP, C), lambda i: (i, 0)),
        out_shape=jax.ShapeDtypeStruct((nrows // K, C), jnp.float32),
    )(q, st2, g2, be2, w2T, b2)


def kernel(feats, inner_w, inner_b, cf_w1, cf_b1, cf_g1, cf_be1, cf_w2,
           cf_b2, mlp_w1, mlp_b1, mlp_g, mlp_be, mlp_w2, mlp_b2):
    del inner_b
    xt3 = jnp.transpose(feats, (0, 2, 1))
    xt_rows = xt3.reshape(B * N, C)
    iw2 = inner_w.reshape(C, 1)

    w1aT = jnp.transpose(cf_w1[:, :C])
    w1bT = jnp.transpose(cf_w1[:, C:])
    b1 = cf_b1.reshape(1, 2 * C)
    wc = jnp.dot(mlp_w1, cf_w2, precision=_PREC)
    bc = (jnp.dot(mlp_w1, cf_b2, precision=_PREC) + mlp_b1).reshape(1, C)
    wcT = jnp.transpose(wc)

    hb = B // 2
    hpts = hb * N
    idxs = [_knn(feats[i * hb:(i + 1) * hb], xt3[i * hb:(i + 1) * hb],
                 i * hb) for i in range(2)]
    Fs = [_sc_gather(xt_rows, idx.reshape(hpts * K)) for idx in idxs]

    cres = [_conv1(F.reshape(hpts, K, C),
                   xt_rows[i * hpts:(i + 1) * hpts], iw2, w1aT, w1bT, b1)
            for i, F in enumerate(Fs)]
    st1 = cres[0][1] + cres[1][1]

    g1r, be1r = cf_g1.reshape(1, 2 * C), cf_be1.reshape(1, 2 * C)
    mres = [_mid(h1, st1, g1r, be1r, wcT, bc) for h1, _ in cres]
    st2 = mres[0][1] + mres[1][1]

    g2r, be2r = mlp_g.reshape(1, C), mlp_be.reshape(1, C)
    w2T, b2r = jnp.transpose(mlp_w2), mlp_b2.reshape(1, C)
    rows = jnp.concatenate(
        [_final(q, st2, g2r, be2r, w2T, b2r) for q, _ in mres], axis=0)
    out = jnp.transpose(rows.reshape(B, N, C), (0, 2, 1))[:, :, :, None]
    return out


# speedup vs baseline: 8.2892x; 1.0155x over previous
import functools

import jax
import jax.numpy as jnp
from jax import lax
from jax.experimental import pallas as pl
from jax.experimental.pallas import tpu as pltpu
from jax.experimental.pallas import tpu_sc as plsc

K = 8
B = 4
C = 128
N = 2048
TN = 256
TP = 128
M = B * N * K
EPS = 1e-5
_PREC = lax.Precision.HIGHEST
_DIST_PREC = lax.Precision.DEFAULT
_CPREC = lax.Precision.DEFAULT


def _knn_body(f_ref, xt_ref, idx_ref, *, boff):
    b = pl.program_id(0)
    x = f_ref[0]
    xt = xt_ref[0]

    xx_full = jnp.sum(x * x, axis=0, keepdims=True)
    xx_row = jnp.sum(xt * xt, axis=1, keepdims=True)
    inner = jnp.dot(xt, x, preferred_element_type=jnp.float32,
                    precision=_DIST_PREC)
    dist = 2.0 * inner - xx_row - xx_full

    col = lax.broadcasted_iota(jnp.int32, (TN, N), 1)
    idx_cols = []
    for _ in range(K):
        idx_j = jnp.argmax(dist, axis=1, keepdims=True)
        dist = jnp.where(col == idx_j, -jnp.inf, dist)
        idx_cols.append(idx_j)
    idx = jnp.concatenate(idx_cols, axis=1)
    idx_ref[0] = idx + (b + boff) * N


def _knn(feats, xt3, boff):
    nb = feats.shape[0]
    grid = (nb, N // TN)
    return pl.pallas_call(
        functools.partial(_knn_body, boff=boff),
        grid=grid,
        in_specs=[
            pl.BlockSpec((1, C, N), lambda b, t: (b, 0, 0)),
            pl.BlockSpec((1, TN, C), lambda b, t: (b, t, 0)),
        ],
        out_specs=pl.BlockSpec((1, TN, K), lambda b, t: (b, t, 0)),
        out_shape=jax.ShapeDtypeStruct((nb, N, K), jnp.int32),
    )(feats, xt3)


_NUM_SC = 2
_NUM_SUBCORES = 16
_NW = _NUM_SC * _NUM_SUBCORES
_CHUNK = 128


def _gather_body(table_hbm, idx_hbm, out_hbm, idx_v, rows_a, rows_b, gsem_a,
                 gsem_b, ssem_a, ssem_b, *, rows_per_w):
    wid = lax.axis_index("c") * _NUM_SUBCORES + lax.axis_index("s")
    base = wid * rows_per_w
    pltpu.sync_copy(idx_hbm.at[pl.ds(base, rows_per_w)], idx_v)

    bufs = (rows_a, rows_b)
    gsems = (gsem_a, gsem_b)
    ssems = (ssem_a, ssem_b)
    nchunk = rows_per_w // _CHUNK

    def gather(c):
        return (table_hbm.at[idx_v.at[pl.ds(c * _CHUNK, _CHUNK)]],
                bufs[c % 2], gsems[c % 2])

    def scatter(c):
        return (bufs[c % 2], out_hbm.at[pl.ds(base + c * _CHUNK, _CHUNK)],
                ssems[c % 2])

    pltpu.async_copy(*gather(0))
    for c in range(nchunk):
        if c + 1 < nchunk:
            if c >= 1:
                pltpu.make_async_copy(*scatter(c - 1)).wait()
            pltpu.async_copy(*gather(c + 1))
        pltpu.make_async_copy(*gather(c)).wait()
        pltpu.async_copy(*scatter(c))
    pltpu.make_async_copy(*scatter(nchunk - 2)).wait()
    pltpu.make_async_copy(*scatter(nchunk - 1)).wait()


def _sc_gather(table, fidx):
    m = fidx.shape[0]
    rows_per_w = m // _NW
    mesh = plsc.VectorSubcoreMesh(core_axis_name="c", subcore_axis_name="s")
    k = pl.kernel(
        functools.partial(_gather_body, rows_per_w=rows_per_w),
        out_type=jax.ShapeDtypeStruct((m, C), jnp.float32),
        mesh=mesh,
        scratch_types=[
            pltpu.VMEM((rows_per_w,), jnp.int32),
            pltpu.VMEM((_CHUNK, C), jnp.float32),
            pltpu.VMEM((_CHUNK, C), jnp.float32),
            pltpu.SemaphoreType.DMA,
            pltpu.SemaphoreType.DMA,
            pltpu.SemaphoreType.DMA,
            pltpu.SemaphoreType.DMA,
        ],
    )
    return k(table, fidx)


def _conv1_body(f_ref, xt_ref, iw_ref, w1a_ref, w1b_ref, b1_ref, h1_ref,
                st_ref):
    i = pl.program_id(0)
    xt = xt_ref[...]
    a = jnp.dot(xt, w1a_ref[...], preferred_element_type=jnp.float32,
                precision=_CPREC) + b1_ref[...]

    ff = f_ref[...].reshape(TP * K, C)
    sg = jnp.dot(ff, iw_ref[...], preferred_element_type=jnp.float32,
                 precision=_PREC)
    sg3 = sg.reshape(TP, K, 1)
    mn = sg3[:, 0, :]
    for j in range(1, K):
        mn = jnp.minimum(mn, sg3[:, j, :])
    e3 = jnp.exp(mn[:, None, :] - sg3)
    den = e3[:, 0, :]
    for j in range(1, K):
        den = den + e3[:, j, :]
    w3 = e3 * (1.0 / den)[:, None, :]

    wf = w3.reshape(TP * K, 1)
    hb = jnp.dot(ff * wf, w1b_ref[...],
                 preferred_element_type=jnp.float32,
                 precision=_CPREC)
    h = hb.reshape(TP, K, 2 * C) + a[:, None, :]
    h1_ref[...] = h.astype(jnp.bfloat16)

    hf = h.reshape(TP * K, 2 * C)
    s1 = jnp.sum(hf, axis=0, keepdims=True)
    s2 = jnp.sum(hf * hf, axis=0, keepdims=True)

    @pl.when(i == 0)
    def _():
        st_ref[...] = jnp.zeros_like(st_ref)

    st_ref[...] += jnp.concatenate([s1, s2], axis=0)


def _conv1(F3, xt_rows, iw2, w1aT, w1bT, b1):
    npts = F3.shape[0]
    grid = (npts // TP,)
    return pl.pallas_call(
        _conv1_body,
        grid=grid,
        in_specs=[
            pl.BlockSpec((TP, K, C), lambda i: (i, 0, 0)),
            pl.BlockSpec((TP, C), lambda i: (i, 0)),
            pl.BlockSpec((C, 1), lambda i: (0, 0)),
            pl.BlockSpec((C, 2 * C), lambda i: (0, 0)),
            pl.BlockSpec((C, 2 * C), lambda i: (0, 0)),
            pl.BlockSpec((1, 2 * C), lambda i: (0, 0)),
        ],
        out_specs=[
            pl.BlockSpec((TP, K, 2 * C), lambda i: (i, 0, 0)),
            pl.BlockSpec((2, 2 * C), lambda i: (0, 0)),
        ],
        out_shape=[
            jax.ShapeDtypeStruct((npts, K, 2 * C), jnp.bfloat16),
            jax.ShapeDtypeStruct((2, 2 * C), jnp.float32),
        ],
    )(F3, xt_rows, iw2, w1aT, w1bT, b1)


def _mid_body(h1_ref, st_ref, g_ref, be_ref, wc_ref, bc_ref, q_ref, st2_ref):
    i = pl.program_id(0)
    st = st_ref[...]
    mean = st[0:1, :] * (1.0 / M)
    var = st[1:2, :] * (1.0 / M) - mean * mean
    inv = lax.rsqrt(var + EPS)
    scale = g_ref[...] * inv
    shift = be_ref[...] - mean * scale

    h = h1_ref[...].astype(jnp.float32).reshape(TP * K, 2 * C)
    h = jnp.maximum(h * scale + shift, 0.0)
    q = jnp.dot(h, wc_ref[...], preferred_element_type=jnp.float32,
                precision=_CPREC) + bc_ref[...]

    @pl.when(i == 0)
    def _():
        st2_ref[...] = jnp.zeros_like(st2_ref)

    s1 = jnp.sum(q, axis=0, keepdims=True)
    s2 = jnp.sum(q * q, axis=0, keepdims=True)
    st2_ref[...] += jnp.concatenate([s1, s2], axis=0)
    q_ref[...] = q.astype(jnp.bfloat16)


def _mid(h1, st1, g1, be1, wcT, bc):
    npts = h1.shape[0]
    grid = (npts // TP,)
    return pl.pallas_call(
        _mid_body,
        grid=grid,
        in_specs=[
            pl.BlockSpec((TP, K, 2 * C), lambda i: (i, 0, 0)),
            pl.BlockSpec((2, 2 * C), lambda i: (0, 0)),
            pl.BlockSpec((1, 2 * C), lambda i: (0, 0)),
            pl.BlockSpec((1, 2 * C), lambda i: (0, 0)),
            pl.BlockSpec((2 * C, C), lambda i: (0, 0)),
            pl.BlockSpec((1, C), lambda i: (0, 0)),
        ],
        out_specs=[
            pl.BlockSpec((TP * K, C), lambda i: (i, 0)),
            pl.BlockSpec((2, C), lambda i: (0, 0)),
        ],
        out_shape=[
            jax.ShapeDtypeStruct((npts * K, C), jnp.bfloat16),
            jax.ShapeDtypeStruct((2, C), jnp.float32),
        ],
    )(h1, st1, g1, be1, wcT, bc)


def _final_body(q_ref, st_ref, g_ref, be_ref, w2_ref, b2_ref, o_ref):
    st = st_ref[...]
    mean = st[0:1, :] * (1.0 / M)
    var = st[1:2, :] * (1.0 / M) - mean * mean
    inv = lax.rsqrt(var + EPS)
    scale = g_ref[...] * inv
    shift = be_ref[...] - mean * scale

    q = jnp.maximum(q_ref[...].astype(jnp.float32) * scale + shift, 0.0)
    z = jnp.dot(q, w2_ref[...], preferred_element_type=jnp.float32,
                precision=_CPREC) + b2_ref[...]
    z3 = z.reshape(TP, K, C)
    m = z3[:, 0, :]
    for j in range(1, K):
        m = jnp.maximum(m, z3[:, j, :])
    o_ref[...] = m


def _final(q, st2, g2, be2, w2T, b2):
    nrows = q.shape[0]
    grid = (nrows // (TP * K),)
    return pl.pallas_call(
        _final_body,
        grid=grid,
        in_specs=[
            pl.BlockSpec((TP * K, C), lambda i: (i, 0)),
            pl.BlockSpec((2, C), lambda i: (0, 0)),
            pl.BlockSpec((1, C), lambda i: (0, 0)),
            pl.BlockSpec((1, C), lambda i: (0, 0)),
            pl.BlockSpec((C, C), lambda i: (0, 0)),
            pl.BlockSpec((1, C), lambda i: (0, 0)),
        ],
        out_specs=pl.BlockSpec((TP, C), lambda i: (i, 0)),
        out_shape=jax.ShapeDtypeStruct((nrows // K, C), jnp.float32),
    )(q, st2, g2, be2, w2T, b2)


def kernel(feats, inner_w, inner_b, cf_w1, cf_b1, cf_g1, cf_be1, cf_w2,
           cf_b2, mlp_w1, mlp_b1, mlp_g, mlp_be, mlp_w2, mlp_b2):
    del inner_b
    xt3 = jnp.transpose(feats, (0, 2, 1))
    xt_rows = xt3.reshape(B * N, C)
    iw2 = inner_w.reshape(C, 1)

    w1aT = jnp.transpose(cf_w1[:, :C])
    w1bT = jnp.transpose(cf_w1[:, C:])
    b1 = cf_b1.reshape(1, 2 * C)
    wc = jnp.dot(mlp_w1, cf_w2, precision=_PREC)
    bc = (jnp.dot(mlp_w1, cf_b2, precision=_PREC) + mlp_b1).reshape(1, C)
    wcT = jnp.transpose(wc)

    hb = B // 2
    hpts = hb * N
    idxs = [_knn(feats[i * hb:(i + 1) * hb], xt3[i * hb:(i + 1) * hb],
                 i * hb) for i in range(2)]
    Fs = [_sc_gather(xt_rows, idx.reshape(hpts * K)) for idx in idxs]

    cres = [_conv1(F.reshape(hpts, K, C),
                   xt_rows[i * hpts:(i + 1) * hpts], iw2, w1aT, w1bT, b1)
            for i, F in enumerate(Fs)]
    st1 = cres[0][1] + cres[1][1]

    g1r, be1r = cf_g1.reshape(1, 2 * C), cf_be1.reshape(1, 2 * C)
    mres = [_mid(h1, st1, g1r, be1r, wcT, bc) for h1, _ in cres]
    st2 = mres[0][1] + mres[1][1]

    g2r, be2r = mlp_g.reshape(1, C), mlp_be.reshape(1, C)
    w2T, b2r = jnp.transpose(mlp_w2), mlp_b2.reshape(1, C)
    rows = jnp.concatenate(
        [_final(q, st2, g2r, be2r, w2T, b2r) for q, _ in mres], axis=0)
    out = jnp.transpose(rows.reshape(B, N, C), (0, 2, 1))[:, :, :, None]
    return out


# speedup vs baseline: 8.2986x; 1.0011x over previous
import functools

import jax
import jax.numpy as jnp
from jax import lax
from jax.experimental import pallas as pl
from jax.experimental.pallas import tpu as pltpu
from jax.experimental.pallas import tpu_sc as plsc

K = 8
B = 4
C = 128
N = 2048
TN = 256
TP = 128
M = B * N * K
EPS = 1e-5
_PREC = lax.Precision.HIGHEST
_DIST_PREC = lax.Precision.DEFAULT
_CPREC = lax.Precision.DEFAULT


def _knn_body(f_ref, xt_ref, idx_ref, *, boff):
    b = pl.program_id(0)
    x = f_ref[0]
    xt = xt_ref[0]

    xx_full = jnp.sum(x * x, axis=0, keepdims=True)
    xx_row = jnp.sum(xt * xt, axis=1, keepdims=True)
    inner = jnp.dot(xt, x, preferred_element_type=jnp.float32,
                    precision=_DIST_PREC)
    dist = 2.0 * inner - xx_row - xx_full

    col = lax.broadcasted_iota(jnp.int32, (TN, N), 1)
    idx_cols = []
    for _ in range(K):
        idx_j = jnp.argmax(dist, axis=1, keepdims=True)
        dist = jnp.where(col == idx_j, -jnp.inf, dist)
        idx_cols.append(idx_j)
    idx = jnp.concatenate(idx_cols, axis=1)
    idx_ref[0] = idx + (b + boff) * N


def _knn(feats, xt3, boff):
    nb = feats.shape[0]
    grid = (nb, N // TN)
    return pl.pallas_call(
        functools.partial(_knn_body, boff=boff),
        grid=grid,
        in_specs=[
            pl.BlockSpec((1, C, N), lambda b, t: (b, 0, 0)),
            pl.BlockSpec((1, TN, C), lambda b, t: (b, t, 0)),
        ],
        out_specs=pl.BlockSpec((1, TN, K), lambda b, t: (b, t, 0)),
        out_shape=jax.ShapeDtypeStruct((nb, N, K), jnp.int32),
    )(feats, xt3)


_NUM_SC = 2
_NUM_SUBCORES = 16
_NW = _NUM_SC * _NUM_SUBCORES
_CHUNK = 256


def _gather_body(table_hbm, idx_hbm, out_hbm, idx_v, rows_a, rows_b, gsem_a,
                 gsem_b, ssem_a, ssem_b, *, rows_per_w):
    wid = lax.axis_index("c") * _NUM_SUBCORES + lax.axis_index("s")
    base = wid * rows_per_w
    pltpu.sync_copy(idx_hbm.at[pl.ds(base, rows_per_w)], idx_v)

    bufs = (rows_a, rows_b)
    gsems = (gsem_a, gsem_b)
    ssems = (ssem_a, ssem_b)
    nchunk = rows_per_w // _CHUNK

    def gather(c):
        return (table_hbm.at[idx_v.at[pl.ds(c * _CHUNK, _CHUNK)]],
                bufs[c % 2], gsems[c % 2])

    def scatter(c):
        return (bufs[c % 2], out_hbm.at[pl.ds(base + c * _CHUNK, _CHUNK)],
                ssems[c % 2])

    pltpu.async_copy(*gather(0))
    for c in range(nchunk):
        if c + 1 < nchunk:
            if c >= 1:
                pltpu.make_async_copy(*scatter(c - 1)).wait()
            pltpu.async_copy(*gather(c + 1))
        pltpu.make_async_copy(*gather(c)).wait()
        pltpu.async_copy(*scatter(c))
    pltpu.make_async_copy(*scatter(nchunk - 2)).wait()
    pltpu.make_async_copy(*scatter(nchunk - 1)).wait()


def _sc_gather(table, fidx):
    m = fidx.shape[0]
    rows_per_w = m // _NW
    mesh = plsc.VectorSubcoreMesh(core_axis_name="c", subcore_axis_name="s")
    k = pl.kernel(
        functools.partial(_gather_body, rows_per_w=rows_per_w),
        out_type=jax.ShapeDtypeStruct((m, C), jnp.float32),
        mesh=mesh,
        scratch_types=[
            pltpu.VMEM((rows_per_w,), jnp.int32),
            pltpu.VMEM((_CHUNK, C), jnp.float32),
            pltpu.VMEM((_CHUNK, C), jnp.float32),
            pltpu.SemaphoreType.DMA,
            pltpu.SemaphoreType.DMA,
            pltpu.SemaphoreType.DMA,
            pltpu.SemaphoreType.DMA,
        ],
    )
    return k(table, fidx)


def _conv1_body(f_ref, xt_ref, iw_ref, w1a_ref, w1b_ref, b1_ref, h1_ref,
                st_ref):
    i = pl.program_id(0)
    xt = xt_ref[...]
    a = jnp.dot(xt, w1a_ref[...], preferred_element_type=jnp.float32,
                precision=_CPREC) + b1_ref[...]

    ff = f_ref[...].reshape(TP * K, C)
    sg = jnp.dot(ff, iw_ref[...], preferred_element_type=jnp.float32,
                 precision=_PREC)
    sg3 = sg.reshape(TP, K, 1)
    mn = sg3[:, 0, :]
    for j in range(1, K):
        mn = jnp.minimum(mn, sg3[:, j, :])
    e3 = jnp.exp(mn[:, None, :] - sg3)
    den = e3[:, 0, :]
    for j in range(1, K):
        den = den + e3[:, j, :]
    w3 = e3 * (1.0 / den)[:, None, :]

    wf = w3.reshape(TP * K, 1)
    hb = jnp.dot(ff * wf, w1b_ref[...],
                 preferred_element_type=jnp.float32,
                 precision=_CPREC)
    h = hb.reshape(TP, K, 2 * C) + a[:, None, :]
    h1_ref[...] = h.astype(jnp.bfloat16)

    hf = h.reshape(TP * K, 2 * C)
    s1 = jnp.sum(hf, axis=0, keepdims=True)
    s2 = jnp.sum(hf * hf, axis=0, keepdims=True)

    @pl.when(i == 0)
    def _():
        st_ref[...] = jnp.zeros_like(st_ref)

    st_ref[...] += jnp.concatenate([s1, s2], axis=0)


def _conv1(F3, xt_rows, iw2, w1aT, w1bT, b1):
    npts = F3.shape[0]
    grid = (npts // TP,)
    return pl.pallas_call(
        _conv1_body,
        grid=grid,
        in_specs=[
            pl.BlockSpec((TP, K, C), lambda i: (i, 0, 0)),
            pl.BlockSpec((TP, C), lambda i: (i, 0)),
            pl.BlockSpec((C, 1), lambda i: (0, 0)),
            pl.BlockSpec((C, 2 * C), lambda i: (0, 0)),
            pl.BlockSpec((C, 2 * C), lambda i: (0, 0)),
            pl.BlockSpec((1, 2 * C), lambda i: (0, 0)),
        ],
        out_specs=[
            pl.BlockSpec((TP, K, 2 * C), lambda i: (i, 0, 0)),
            pl.BlockSpec((2, 2 * C), lambda i: (0, 0)),
        ],
        out_shape=[
            jax.ShapeDtypeStruct((npts, K, 2 * C), jnp.bfloat16),
            jax.ShapeDtypeStruct((2, 2 * C), jnp.float32),
        ],
    )(F3, xt_rows, iw2, w1aT, w1bT, b1)


def _mid_body(h1_ref, st_ref, g_ref, be_ref, wc_ref, bc_ref, q_ref, st2_ref):
    i = pl.program_id(0)
    st = st_ref[...]
    mean = st[0:1, :] * (1.0 / M)
    var = st[1:2, :] * (1.0 / M) - mean * mean
    inv = lax.rsqrt(var + EPS)
    scale = g_ref[...] * inv
    shift = be_ref[...] - mean * scale

    h = h1_ref[...].astype(jnp.float32).reshape(TP * K, 2 * C)
    h = jnp.maximum(h * scale + shift, 0.0)
    q = jnp.dot(h, wc_ref[...], preferred_element_type=jnp.float32,
                precision=_CPREC) + bc_ref[...]

    @pl.when(i == 0)
    def _():
        st2_ref[...] = jnp.zeros_like(st2_ref)

    s1 = jnp.sum(q, axis=0, keepdims=True)
    s2 = jnp.sum(q * q, axis=0, keepdims=True)
    st2_ref[...] += jnp.concatenate([s1, s2], axis=0)
    q_ref[...] = q.astype(jnp.bfloat16)


def _mid(h1, st1, g1, be1, wcT, bc):
    npts = h1.shape[0]
    grid = (npts // TP,)
    return pl.pallas_call(
        _mid_body,
        grid=grid,
        in_specs=[
            pl.BlockSpec((TP, K, 2 * C), lambda i: (i, 0, 0)),
            pl.BlockSpec((2, 2 * C), lambda i: (0, 0)),
            pl.BlockSpec((1, 2 * C), lambda i: (0, 0)),
            pl.BlockSpec((1, 2 * C), lambda i: (0, 0)),
            pl.BlockSpec((2 * C, C), lambda i: (0, 0)),
            pl.BlockSpec((1, C), lambda i: (0, 0)),
        ],
        out_specs=[
            pl.BlockSpec((TP * K, C), lambda i: (i, 0)),
            pl.BlockSpec((2, C), lambda i: (0, 0)),
        ],
        out_shape=[
            jax.ShapeDtypeStruct((npts * K, C), jnp.bfloat16),
            jax.ShapeDtypeStruct((2, C), jnp.float32),
        ],
    )(h1, st1, g1, be1, wcT, bc)


def _final_body(q_ref, st_ref, g_ref, be_ref, w2_ref, b2_ref, o_ref):
    st = st_ref[...]
    mean = st[0:1, :] * (1.0 / M)
    var = st[1:2, :] * (1.0 / M) - mean * mean
    inv = lax.rsqrt(var + EPS)
    scale = g_ref[...] * inv
    shift = be_ref[...] - mean * scale

    q = jnp.maximum(q_ref[...].astype(jnp.float32) * scale + shift, 0.0)
    z = jnp.dot(q, w2_ref[...], preferred_element_type=jnp.float32,
                precision=_CPREC) + b2_ref[...]
    z3 = z.reshape(TP, K, C)
    m = z3[:, 0, :]
    for j in range(1, K):
        m = jnp.maximum(m, z3[:, j, :])
    o_ref[...] = m


def _final(q, st2, g2, be2, w2T, b2):
    nrows = q.shape[0]
    grid = (nrows // (TP * K),)
    return pl.pallas_call(
        _final_body,
        grid=grid,
        in_specs=[
            pl.BlockSpec((TP * K, C), lambda i: (i, 0)),
            pl.BlockSpec((2, C), lambda i: (0, 0)),
            pl.BlockSpec((1, C), lambda i: (0, 0)),
            pl.BlockSpec((1, C), lambda i: (0, 0)),
            pl.BlockSpec((C, C), lambda i: (0, 0)),
            pl.BlockSpec((1, C), lambda i: (0, 0)),
        ],
        out_specs=pl.BlockSpec((TP, C), lambda i: (i, 0)),
        out_shape=jax.ShapeDtypeStruct((nrows // K, C), jnp.float32),
    )(q, st2, g2, be2, w2T, b2)


def kernel(feats, inner_w, inner_b, cf_w1, cf_b1, cf_g1, cf_be1, cf_w2,
           cf_b2, mlp_w1, mlp_b1, mlp_g, mlp_be, mlp_w2, mlp_b2):
    del inner_b
    xt3 = jnp.transpose(feats, (0, 2, 1))
    xt_rows = xt3.reshape(B * N, C)
    iw2 = inner_w.reshape(C, 1)

    w1aT = jnp.transpose(cf_w1[:, :C])
    w1bT = jnp.transpose(cf_w1[:, C:])
    b1 = cf_b1.reshape(1, 2 * C)
    wc = jnp.dot(mlp_w1, cf_w2, precision=_PREC)
    bc = (jnp.dot(mlp_w1, cf_b2, precision=_PREC) + mlp_b1).reshape(1, C)
    wcT = jnp.transpose(wc)

    hb = B // 2
    hpts = hb * N
    idxs = [_knn(feats[i * hb:(i + 1) * hb], xt3[i * hb:(i + 1) * hb],
                 i * hb) for i in range(2)]
    Fs = [_sc_gather(xt_rows, idx.reshape(hpts * K)) for idx in idxs]

    cres = [_conv1(F.reshape(hpts, K, C),
                   xt_rows[i * hpts:(i + 1) * hpts], iw2, w1aT, w1bT, b1)
            for i, F in enumerate(Fs)]
    st1 = cres[0][1] + cres[1][1]

    g1r, be1r = cf_g1.reshape(1, 2 * C), cf_be1.reshape(1, 2 * C)
    mres = [_mid(h1, st1, g1r, be1r, wcT, bc) for h1, _ in cres]
    st2 = mres[0][1] + mres[1][1]

    g2r, be2r = mlp_g.reshape(1, C), mlp_be.reshape(1, C)
    w2T, b2r = jnp.transpose(mlp_w2), mlp_b2.reshape(1, C)
    rows = jnp.concatenate(
        [_final(q, st2, g2r, be2r, w2T, b2r) for q, _ in mres], axis=0)
    out = jnp.transpose(rows.reshape(B, N, C), (0, 2, 1))[:, :, :, None]
    return out
